# baseline (speedup 1.0000x reference)
import functools

import jax
import jax.numpy as jnp
from jax.experimental import pallas as pl
from jax.experimental.pallas import tpu as pltpu
from jax.experimental.pallas import tpu_sc as plsc

T = 4096
H = 2048
I = 1024
E = 16
TOPK = 2
NCH = H // 128

BM = 256
RBM = 512
BMC = 512
GW = 256
S_BUF = 8192 + E * BM
ROUTED_BLOCKS = S_BUF // BM
NEG = -1.7e38


def _router_body(x_ref, gwt_ref, bias_ref, o_ref, xch_ref):
    xb = x_ref[...]
    logits = jax.lax.dot_general(
        xb.astype(jnp.bfloat16), gwt_ref[...].astype(jnp.bfloat16),
        (((1,), (0,)), ((), ())), preferred_element_type=jnp.float32,
    ) + bias_ref[...]
    for k in range(NCH):
        xch_ref[k] = xb[:, 128 * k:128 * (k + 1)]
    lane = jax.lax.broadcasted_iota(jnp.int32, logits.shape, 1)
    m1 = jnp.max(logits, axis=1, keepdims=True)
    a1 = jnp.argmax(logits, axis=1).astype(jnp.int32)
    masked = jnp.where(lane == a1[:, None], NEG, logits)
    m2 = jnp.max(masked, axis=1, keepdims=True)
    a2 = jnp.argmax(masked, axis=1).astype(jnp.int32)
    w1 = 1.0 / (1.0 + jnp.exp(m2 - m1))
    o_ref[...] = (jnp.where(lane == 0, a1[:, None].astype(jnp.float32), 0.0)
                  + jnp.where(lane == 1, a2[:, None].astype(jnp.float32), 0.0)
                  + jnp.where(lane == 2, w1, 0.0)
                  + jnp.where(lane == 3, 1.0 - w1, 0.0))


def _router(x, gate_W, bias):
    gwt = jnp.zeros((H, 128), jnp.float32).at[:, :E].set(gate_W.T)
    bias_row = jnp.full((1, 128), NEG, jnp.float32).at[0, :E].set(bias)
    return pl.pallas_call(
        _router_body,
        grid=(T // RBM,),
        in_specs=[
            pl.BlockSpec((RBM, H), lambda i: (i, 0)),
            pl.BlockSpec((H, 128), lambda i: (0, 0)),
            pl.BlockSpec((1, 128), lambda i: (0, 0)),
        ],
        out_specs=[pl.BlockSpec((RBM, 128), lambda i: (i, 0)),
                   pl.BlockSpec((NCH, RBM, 128), lambda i: (0, i, 0))],
        out_shape=[jax.ShapeDtypeStruct((T, 128), jnp.float32),
                   jax.ShapeDtypeStruct((NCH, T, 128), jnp.float32)],
    )(x, gwt, bias_row)


def _sc_gather_chunks(src_ch, idx):
    nch, nsrc, _ = src_ch.shape
    n = idx.shape[0]
    src2 = src_ch.reshape(nch * nsrc, 128)
    j_idx = (jnp.arange(nch, dtype=jnp.int32)[:, None] * nsrc
             + idx[None, :]).reshape(1, nch * n)
    nsteps = (nch * n) // GW

    @functools.partial(
        pl.kernel,
        out_type=jax.ShapeDtypeStruct((nch * n, 128), src_ch.dtype),
        mesh=plsc.VectorSubcoreMesh(core_axis_name="core",
                                    subcore_axis_name="subcore"),
    )
    def k(x_hbm, i_hbm, o_hbm):
        def body(i_vmem, o_vmem):
            pltpu.sync_copy(x_hbm.at[i_vmem.at[0]], o_vmem)

        pltpu.emit_pipeline(
            body,
            grid=(nsteps,),
            in_specs=[pl.BlockSpec((1, GW), lambda i: (0, i))],
            out_specs=[pl.BlockSpec((GW, 128), lambda i: (i, 0))],
            core_axis_name=("core", "subcore"),
            dimension_semantics=(pltpu.PARALLEL,),
        )(i_hbm, o_hbm)

    return k(src2, j_idx).reshape(nch, n, 128)


def _gmm1_body(be_ref, x_ref, w_ref, o_ref):
    wb = w_ref[0].astype(jnp.bfloat16)
    g = None
    u = None
    for k in range(NCH):
        xk = x_ref[k].astype(jnp.bfloat16)
        gk = jax.lax.dot_general(
            xk, wb[:I, 128 * k:128 * (k + 1)], (((1,), (1,)), ((), ())),
            preferred_element_type=jnp.float32)
        uk = jax.lax.dot_general(
            xk, wb[I:, 128 * k:128 * (k + 1)], (((1,), (1,)), ((), ())),
            preferred_element_type=jnp.float32)
        g = gk if g is None else g + gk
        u = uk if u is None else u + uk
    o_ref[...] = (g * jax.nn.sigmoid(g) * u).astype(jnp.bfloat16)


def _gmm2_body(be_ref, a_ref, w_ref, o_ref):
    ab = a_ref[...]
    wb = w_ref[0].astype(jnp.bfloat16)
    for k in range(NCH):
        o_ref[k] = jax.lax.dot_general(
            ab, wb[128 * k:128 * (k + 1), :], (((1,), (1,)), ((), ())),
            preferred_element_type=jnp.float32)


def _gmm1(x_ch, block_expert, w13):
    s = x_ch.shape[1]
    return pl.pallas_call(
        _gmm1_body,
        grid_spec=pltpu.PrefetchScalarGridSpec(
            num_scalar_prefetch=1,
            grid=(s // BM,),
            in_specs=[
                pl.BlockSpec((NCH, BM, 128), lambda i, be: (0, i, 0)),
                pl.BlockSpec((1, 2 * I, H), lambda i, be: (be[i], 0, 0)),
            ],
            out_specs=pl.BlockSpec((BM, I), lambda i, be: (i, 0)),
        ),
        out_shape=jax.ShapeDtypeStruct((s, I), jnp.bfloat16),
    )(block_expert, x_ch, w13)


def _gmm2(act, block_expert, w2):
    s = act.shape[0]
    return pl.pallas_call(
        _gmm2_body,
        grid_spec=pltpu.PrefetchScalarGridSpec(
            num_scalar_prefetch=1,
            grid=(s // BM,),
            in_specs=[
                pl.BlockSpec((BM, I), lambda i, be: (i, 0)),
                pl.BlockSpec((1, H, I), lambda i, be: (be[i], 0, 0)),
            ],
            out_specs=pl.BlockSpec((NCH, BM, 128), lambda i, be: (0, i, 0)),
        ),
        out_shape=jax.ShapeDtypeStruct((NCH, s, 128), jnp.float32),
    )(block_expert, act, w2)


def _combine_body(ds_ref, da_ref, db_ref, r_ref, o_ref):
    w0 = r_ref[:, 2:3]
    w1 = r_ref[:, 3:4]
    for k in range(NCH):
        o_ref[:, 128 * k:128 * (k + 1)] = (
            ds_ref[k] + w0 * da_ref[k] + w1 * db_ref[k])


def _combine(ds_ch, d01_ch, router_out):
    return pl.pallas_call(
        _combine_body,
        grid=(T // BMC,),
        in_specs=[
            pl.BlockSpec((NCH, BMC, 128), lambda i: (0, i, 0)),
            pl.BlockSpec((NCH, BMC, 128), lambda i: (0, i, 0)),
            pl.BlockSpec((NCH, BMC, 128), lambda i: (0, i + T // BMC, 0)),
            pl.BlockSpec((BMC, 128), lambda i: (i, 0)),
        ],
        out_specs=pl.BlockSpec((BMC, H), lambda i: (i, 0)),
        out_shape=jax.ShapeDtypeStruct((T, H), jnp.float32),
    )(ds_ch, d01_ch, d01_ch, router_out)


def kernel(hidden_states, gate_W, e_score_correction_bias, expert_w13,
           expert_w2, shared_w13, shared_w2):
    x = hidden_states

    router_out, x_ch = _router(x, gate_W, e_score_correction_bias)
    ids = router_out[:, :TOPK].astype(jnp.int32)

    e_flat = ids.reshape(-1)
    onehot = (e_flat[:, None] == jnp.arange(E)[None, :]).astype(jnp.int32)
    csum = jnp.cumsum(onehot, axis=0)
    rank = jnp.take_along_axis(csum - onehot, e_flat[:, None], axis=1)[:, 0]
    counts = csum[-1]
    padded = ((counts + BM - 1) // BM) * BM
    pcum = jnp.cumsum(padded)
    poff = pcum - padded
    pos = poff[e_flat] + rank
    tok_of_slot = jnp.arange(T * TOPK, dtype=jnp.int32) // TOPK
    buf_tok = jnp.zeros((S_BUF,), jnp.int32).at[pos].set(
        tok_of_slot, unique_indices=True)
    block_expert = jnp.minimum(
        jnp.searchsorted(pcum, jnp.arange(ROUTED_BLOCKS) * BM, side="right"),
        E - 1).astype(jnp.int32)

    x_buf_ch = _sc_gather_chunks(x_ch, buf_tok)

    act = _gmm1(x_buf_ch, block_expert, expert_w13)
    down_ch = _gmm2(act, block_expert, expert_w2)

    shared_be = jnp.zeros((T // BM,), jnp.int32)
    ds_ch = _gmm2(_gmm1(x_ch, shared_be, shared_w13[None]),
                  shared_be, shared_w2[None])

    pos_cat = jnp.concatenate([pos[0::TOPK], pos[1::TOPK]])
    d01_ch = _sc_gather_chunks(down_ch, pos_cat)
    return _combine(ds_ch, d01_ch, router_out)


# speedup vs baseline: 1.1414x; 1.1414x over previous
import functools

import jax
import jax.numpy as jnp
from jax.experimental import pallas as pl
from jax.experimental.pallas import tpu as pltpu
from jax.experimental.pallas import tpu_sc as plsc

T = 4096
H = 2048
I = 1024
E = 16
TOPK = 2
NCH = H // 128

BM = 256
RBM = 512
BMC = 512
GW = 256
S_BUF = 8192 + E * BM
ROUTED_BLOCKS = S_BUF // BM
NEG = -1.7e38


def _router_body(x_ref, gwt_ref, bias_ref, o_ref, xch_ref):
    xb = x_ref[...]
    logits = jax.lax.dot_general(
        xb.astype(jnp.bfloat16), gwt_ref[...].astype(jnp.bfloat16),
        (((1,), (0,)), ((), ())), preferred_element_type=jnp.float32,
    ) + bias_ref[...]
    for k in range(NCH):
        xch_ref[k] = xb[:, 128 * k:128 * (k + 1)]
    lane = jax.lax.broadcasted_iota(jnp.int32, logits.shape, 1)
    m1 = jnp.max(logits, axis=1, keepdims=True)
    a1 = jnp.argmax(logits, axis=1).astype(jnp.int32)
    masked = jnp.where(lane == a1[:, None], NEG, logits)
    m2 = jnp.max(masked, axis=1, keepdims=True)
    a2 = jnp.argmax(masked, axis=1).astype(jnp.int32)
    w1 = 1.0 / (1.0 + jnp.exp(m2 - m1))
    o_ref[...] = (jnp.where(lane == 0, a1[:, None].astype(jnp.float32), 0.0)
                  + jnp.where(lane == 1, a2[:, None].astype(jnp.float32), 0.0)
                  + jnp.where(lane == 2, w1, 0.0)
                  + jnp.where(lane == 3, 1.0 - w1, 0.0))


def _router(x, gate_W, bias):
    gwt = jnp.zeros((H, 128), jnp.float32).at[:, :E].set(gate_W.T)
    bias_row = jnp.full((1, 128), NEG, jnp.float32).at[0, :E].set(bias)
    return pl.pallas_call(
        _router_body,
        grid=(T // RBM,),
        in_specs=[
            pl.BlockSpec((RBM, H), lambda i: (i, 0)),
            pl.BlockSpec((H, 128), lambda i: (0, 0)),
            pl.BlockSpec((1, 128), lambda i: (0, 0)),
        ],
        out_specs=[pl.BlockSpec((RBM, 128), lambda i: (i, 0)),
                   pl.BlockSpec((NCH, RBM, 128), lambda i: (0, i, 0))],
        out_shape=[jax.ShapeDtypeStruct((T, 128), jnp.float32),
                   jax.ShapeDtypeStruct((NCH, T, 128), jnp.float32)],
    )(x, gwt, bias_row)


def _sc_gather_chunks(src_ch, idx):
    nch, nsrc, _ = src_ch.shape
    n = idx.shape[0]
    src2 = src_ch.reshape(nch * nsrc, 128)
    j_idx = (jnp.arange(nch, dtype=jnp.int32)[:, None] * nsrc
             + idx[None, :]).reshape(1, nch * n)
    nsteps = (nch * n) // GW

    @functools.partial(
        pl.kernel,
        out_type=jax.ShapeDtypeStruct((nch * n, 128), src_ch.dtype),
        mesh=plsc.VectorSubcoreMesh(core_axis_name="core",
                                    subcore_axis_name="subcore"),
    )
    def k(x_hbm, i_hbm, o_hbm):
        def body(i_vmem, o_vmem):
            pltpu.sync_copy(x_hbm.at[i_vmem.at[0]], o_vmem)

        pltpu.emit_pipeline(
            body,
            grid=(nsteps,),
            in_specs=[pl.BlockSpec((1, GW), lambda i: (0, i))],
            out_specs=[pl.BlockSpec((GW, 128), lambda i: (i, 0))],
            core_axis_name=("core", "subcore"),
            dimension_semantics=(pltpu.PARALLEL,),
        )(i_hbm, o_hbm)

    return k(src2, j_idx).reshape(nch, n, 128)


def _gmm1_body(be_ref, x_ref, w_ref, o_ref):
    wb = w_ref[0].astype(jnp.bfloat16)
    xf = jnp.concatenate(
        [x_ref[k].astype(jnp.bfloat16) for k in range(NCH)], axis=1)
    g = jax.lax.dot_general(xf, wb[:I, :], (((1,), (1,)), ((), ())),
                            preferred_element_type=jnp.float32)
    u = jax.lax.dot_general(xf, wb[I:, :], (((1,), (1,)), ((), ())),
                            preferred_element_type=jnp.float32)
    o_ref[...] = (g * jax.nn.sigmoid(g) * u).astype(jnp.bfloat16)


def _gmm2_body(be_ref, a_ref, w_ref, o_ref):
    down = jax.lax.dot_general(
        a_ref[...], w_ref[0].astype(jnp.bfloat16),
        (((1,), (1,)), ((), ())), preferred_element_type=jnp.float32)
    for k in range(NCH):
        o_ref[k] = down[:, 128 * k:128 * (k + 1)]


def _gmm1(x_ch, block_expert, w13):
    s = x_ch.shape[1]
    return pl.pallas_call(
        _gmm1_body,
        grid_spec=pltpu.PrefetchScalarGridSpec(
            num_scalar_prefetch=1,
            grid=(s // BM,),
            in_specs=[
                pl.BlockSpec((NCH, BM, 128), lambda i, be: (0, i, 0)),
                pl.BlockSpec((1, 2 * I, H), lambda i, be: (be[i], 0, 0)),
            ],
            out_specs=pl.BlockSpec((BM, I), lambda i, be: (i, 0)),
        ),
        out_shape=jax.ShapeDtypeStruct((s, I), jnp.bfloat16),
    )(block_expert, x_ch, w13)


def _gmm2(act, block_expert, w2):
    s = act.shape[0]
    return pl.pallas_call(
        _gmm2_body,
        grid_spec=pltpu.PrefetchScalarGridSpec(
            num_scalar_prefetch=1,
            grid=(s // BM,),
            in_specs=[
                pl.BlockSpec((BM, I), lambda i, be: (i, 0)),
                pl.BlockSpec((1, H, I), lambda i, be: (be[i], 0, 0)),
            ],
            out_specs=pl.BlockSpec((NCH, BM, 128), lambda i, be: (0, i, 0)),
        ),
        out_shape=jax.ShapeDtypeStruct((NCH, s, 128), jnp.float32),
    )(block_expert, act, w2)


def _combine_body(ds_ref, da_ref, db_ref, r_ref, o_ref):
    w0 = r_ref[:, 2:3]
    w1 = r_ref[:, 3:4]
    for k in range(NCH):
        o_ref[:, 128 * k:128 * (k + 1)] = (
            ds_ref[k] + w0 * da_ref[k] + w1 * db_ref[k])


def _combine(ds_ch, d01_ch, router_out):
    return pl.pallas_call(
        _combine_body,
        grid=(T // BMC,),
        in_specs=[
            pl.BlockSpec((NCH, BMC, 128), lambda i: (0, i, 0)),
            pl.BlockSpec((NCH, BMC, 128), lambda i: (0, i, 0)),
            pl.BlockSpec((NCH, BMC, 128), lambda i: (0, i + T // BMC, 0)),
            pl.BlockSpec((BMC, 128), lambda i: (i, 0)),
        ],
        out_specs=pl.BlockSpec((BMC, H), lambda i: (i, 0)),
        out_shape=jax.ShapeDtypeStruct((T, H), jnp.float32),
    )(ds_ch, d01_ch, d01_ch, router_out)


def kernel(hidden_states, gate_W, e_score_correction_bias, expert_w13,
           expert_w2, shared_w13, shared_w2):
    x = hidden_states

    router_out, x_ch = _router(x, gate_W, e_score_correction_bias)
    ids = router_out[:, :TOPK].astype(jnp.int32)

    e_flat = ids.reshape(-1)
    onehot = (e_flat[:, None] == jnp.arange(E)[None, :]).astype(jnp.int32)
    csum = jnp.cumsum(onehot, axis=0)
    rank = jnp.take_along_axis(csum - onehot, e_flat[:, None], axis=1)[:, 0]
    counts = csum[-1]
    padded = ((counts + BM - 1) // BM) * BM
    pcum = jnp.cumsum(padded)
    poff = pcum - padded
    pos = poff[e_flat] + rank
    tok_of_slot = jnp.arange(T * TOPK, dtype=jnp.int32) // TOPK
    buf_tok = jnp.zeros((S_BUF,), jnp.int32).at[pos].set(
        tok_of_slot, unique_indices=True)
    block_expert = jnp.minimum(
        jnp.searchsorted(pcum, jnp.arange(ROUTED_BLOCKS) * BM, side="right"),
        E - 1).astype(jnp.int32)

    shared_be = jnp.zeros((T // BM,), jnp.int32)
    ds_ch = _gmm2(_gmm1(x_ch, shared_be, shared_w13[None]),
                  shared_be, shared_w2[None])

    half = S_BUF // 2
    hb = ROUTED_BLOCKS // 2
    xba = _sc_gather_chunks(x_ch, buf_tok[:half])
    xbb = _sc_gather_chunks(x_ch, buf_tok[half:])

    act = jnp.concatenate(
        [_gmm1(xba, block_expert[:hb], expert_w13),
         _gmm1(xbb, block_expert[hb:], expert_w13)], axis=0)
    down_ch = _gmm2(act, block_expert, expert_w2)

    pos_cat = jnp.concatenate([pos[0::TOPK], pos[1::TOPK]])
    d01_ch = _sc_gather_chunks(down_ch, pos_cat)
    return _combine(ds_ch, d01_ch, router_out)


# speedup vs baseline: 1.2255x; 1.0737x over previous
import functools

import jax
import jax.numpy as jnp
from jax.experimental import pallas as pl
from jax.experimental.pallas import tpu as pltpu
from jax.experimental.pallas import tpu_sc as plsc

T = 4096
H = 2048
I = 1024
E = 16
TOPK = 2
NCH = H // 128
NPK = NCH // 2

BM = 256
RBM = 512
BMC = 512
GW = 256
S_BUF = 8192 + E * BM
ROUTED_BLOCKS = S_BUF // BM
NEG = -1.7e38


def _router_body(x_ref, gwt_ref, bias_ref, o_ref, xch_ref):
    xb = x_ref[...]
    logits = jax.lax.dot_general(
        xb.astype(jnp.bfloat16), gwt_ref[...].astype(jnp.bfloat16),
        (((1,), (0,)), ((), ())), preferred_element_type=jnp.float32,
    ) + bias_ref[...]
    xbb = xb.astype(jnp.bfloat16)
    for j in range(NPK):
        lo = jax.lax.bitcast_convert_type(
            xbb[:, 256 * j:256 * j + 128].astype(jnp.float32), jnp.uint32)
        hi = jax.lax.bitcast_convert_type(
            xbb[:, 256 * j + 128:256 * j + 256].astype(jnp.float32),
            jnp.uint32)
        xch_ref[j] = (hi & jnp.uint32(0xFFFF0000)) | (lo >> 16)
    lane = jax.lax.broadcasted_iota(jnp.int32, logits.shape, 1)
    m1 = jnp.max(logits, axis=1, keepdims=True)
    a1 = jnp.argmax(logits, axis=1).astype(jnp.int32)
    masked = jnp.where(lane == a1[:, None], NEG, logits)
    m2 = jnp.max(masked, axis=1, keepdims=True)
    a2 = jnp.argmax(masked, axis=1).astype(jnp.int32)
    w1 = 1.0 / (1.0 + jnp.exp(m2 - m1))
    o_ref[...] = (jnp.where(lane == 0, a1[:, None].astype(jnp.float32), 0.0)
                  + jnp.where(lane == 1, a2[:, None].astype(jnp.float32), 0.0)
                  + jnp.where(lane == 2, w1, 0.0)
                  + jnp.where(lane == 3, 1.0 - w1, 0.0))


def _router(x, gate_W, bias):
    gwt = jnp.zeros((H, 128), jnp.float32).at[:, :E].set(gate_W.T)
    bias_row = jnp.full((1, 128), NEG, jnp.float32).at[0, :E].set(bias)
    return pl.pallas_call(
        _router_body,
        grid=(T // RBM,),
        in_specs=[
            pl.BlockSpec((RBM, H), lambda i: (i, 0)),
            pl.BlockSpec((H, 128), lambda i: (0, 0)),
            pl.BlockSpec((1, 128), lambda i: (0, 0)),
        ],
        out_specs=[pl.BlockSpec((RBM, 128), lambda i: (i, 0)),
                   pl.BlockSpec((NPK, RBM, 128), lambda i: (0, i, 0))],
        out_shape=[jax.ShapeDtypeStruct((T, 128), jnp.float32),
                   jax.ShapeDtypeStruct((NPK, T, 128), jnp.uint32)],
    )(x, gwt, bias_row)


def _sc_gather_chunks(src_ch, idx):
    nch, nsrc, _ = src_ch.shape
    n = idx.shape[0]
    src2 = src_ch.reshape(nch * nsrc, 128)
    j_idx = (jnp.arange(nch, dtype=jnp.int32)[:, None] * nsrc
             + idx[None, :]).reshape(1, nch * n)
    nsteps = (nch * n) // GW

    @functools.partial(
        pl.kernel,
        out_type=jax.ShapeDtypeStruct((nch * n, 128), src_ch.dtype),
        mesh=plsc.VectorSubcoreMesh(core_axis_name="core",
                                    subcore_axis_name="subcore"),
    )
    def k(x_hbm, i_hbm, o_hbm):
        def body(i_vmem, o_vmem):
            pltpu.sync_copy(x_hbm.at[i_vmem.at[0]], o_vmem)

        pltpu.emit_pipeline(
            body,
            grid=(nsteps,),
            in_specs=[pl.BlockSpec((1, GW), lambda i: (0, i))],
            out_specs=[pl.BlockSpec((GW, 128), lambda i: (i, 0))],
            core_axis_name=("core", "subcore"),
            dimension_semantics=(pltpu.PARALLEL,),
        )(i_hbm, o_hbm)

    return k(src2, j_idx).reshape(nch, n, 128)


def _gmm1_body(be_ref, x_ref, w_ref, o_ref):
    wb = w_ref[0].astype(jnp.bfloat16)
    chunks = []
    for j in range(NPK):
        w = x_ref[j]
        chunks.append(jax.lax.bitcast_convert_type(w << 16, jnp.float32))
        chunks.append(jax.lax.bitcast_convert_type(
            w & jnp.uint32(0xFFFF0000), jnp.float32))
    xf = jnp.concatenate(chunks, axis=1).astype(jnp.bfloat16)
    g = jax.lax.dot_general(xf, wb[:I, :], (((1,), (1,)), ((), ())),
                            preferred_element_type=jnp.float32)
    u = jax.lax.dot_general(xf, wb[I:, :], (((1,), (1,)), ((), ())),
                            preferred_element_type=jnp.float32)
    o_ref[...] = (g * jax.nn.sigmoid(g) * u).astype(jnp.bfloat16)


def _gmm2_body(be_ref, a_ref, w_ref, o_ref):
    down = jax.lax.dot_general(
        a_ref[...], w_ref[0].astype(jnp.bfloat16),
        (((1,), (1,)), ((), ())), preferred_element_type=jnp.float32)
    for k in range(NCH):
        o_ref[k] = down[:, 128 * k:128 * (k + 1)]


def _gmm1(x_ch, block_expert, w13):
    s = x_ch.shape[1]
    return pl.pallas_call(
        _gmm1_body,
        grid_spec=pltpu.PrefetchScalarGridSpec(
            num_scalar_prefetch=1,
            grid=(s // BM,),
            in_specs=[
                pl.BlockSpec((NPK, BM, 128), lambda i, be: (0, i, 0)),
                pl.BlockSpec((1, 2 * I, H), lambda i, be: (be[i], 0, 0)),
            ],
            out_specs=pl.BlockSpec((BM, I), lambda i, be: (i, 0)),
        ),
        out_shape=jax.ShapeDtypeStruct((s, I), jnp.bfloat16),
    )(block_expert, x_ch, w13)


def _gmm2(act, block_expert, w2):
    s = act.shape[0]
    return pl.pallas_call(
        _gmm2_body,
        grid_spec=pltpu.PrefetchScalarGridSpec(
            num_scalar_prefetch=1,
            grid=(s // BM,),
            in_specs=[
                pl.BlockSpec((BM, I), lambda i, be: (i, 0)),
                pl.BlockSpec((1, H, I), lambda i, be: (be[i], 0, 0)),
            ],
            out_specs=pl.BlockSpec((NCH, BM, 128), lambda i, be: (0, i, 0)),
        ),
        out_shape=jax.ShapeDtypeStruct((NCH, s, 128), jnp.float32),
    )(block_expert, act, w2)


def _combine_body(ds_ref, da_ref, db_ref, r_ref, o_ref):
    w0 = r_ref[:, 2:3]
    w1 = r_ref[:, 3:4]
    for k in range(NCH):
        o_ref[:, 128 * k:128 * (k + 1)] = (
            ds_ref[k] + w0 * da_ref[k] + w1 * db_ref[k])


def _combine(ds_ch, d01_ch, router_out):
    return pl.pallas_call(
        _combine_body,
        grid=(T // BMC,),
        in_specs=[
            pl.BlockSpec((NCH, BMC, 128), lambda i: (0, i, 0)),
            pl.BlockSpec((NCH, BMC, 128), lambda i: (0, i, 0)),
            pl.BlockSpec((NCH, BMC, 128), lambda i: (0, i + T // BMC, 0)),
            pl.BlockSpec((BMC, 128), lambda i: (i, 0)),
        ],
        out_specs=pl.BlockSpec((BMC, H), lambda i: (i, 0)),
        out_shape=jax.ShapeDtypeStruct((T, H), jnp.float32),
    )(ds_ch, d01_ch, d01_ch, router_out)


def kernel(hidden_states, gate_W, e_score_correction_bias, expert_w13,
           expert_w2, shared_w13, shared_w2):
    x = hidden_states

    router_out, x_ch = _router(x, gate_W, e_score_correction_bias)
    ids = router_out[:, :TOPK].astype(jnp.int32)

    e_flat = ids.reshape(-1)
    onehot = (e_flat[:, None] == jnp.arange(E)[None, :]).astype(jnp.int32)
    csum = jnp.cumsum(onehot, axis=0)
    rank = jnp.take_along_axis(csum - onehot, e_flat[:, None], axis=1)[:, 0]
    counts = csum[-1]
    padded = ((counts + BM - 1) // BM) * BM
    pcum = jnp.cumsum(padded)
    poff = pcum - padded
    pos = poff[e_flat] + rank
    tok_of_slot = jnp.arange(T * TOPK, dtype=jnp.int32) // TOPK
    buf_tok = jnp.zeros((S_BUF,), jnp.int32).at[pos].set(
        tok_of_slot, unique_indices=True)
    block_expert = jnp.minimum(
        jnp.searchsorted(pcum, jnp.arange(ROUTED_BLOCKS) * BM, side="right"),
        E - 1).astype(jnp.int32)

    shared_be = jnp.zeros((T // BM,), jnp.int32)
    ds_ch = _gmm2(_gmm1(x_ch, shared_be, shared_w13[None]),
                  shared_be, shared_w2[None])

    half = S_BUF // 2
    hb = ROUTED_BLOCKS // 2
    xba = _sc_gather_chunks(x_ch, buf_tok[:half])
    xbb = _sc_gather_chunks(x_ch, buf_tok[half:])

    act = jnp.concatenate(
        [_gmm1(xba, block_expert[:hb], expert_w13),
         _gmm1(xbb, block_expert[hb:], expert_w13)], axis=0)
    down_ch = _gmm2(act, block_expert, expert_w2)

    pos_cat = jnp.concatenate([pos[0::TOPK], pos[1::TOPK]])
    d01_ch = _sc_gather_chunks(down_ch, pos_cat)
    return _combine(ds_ch, d01_ch, router_out)
